# baseline (speedup 1.0000x reference)
import functools

import jax
import jax.numpy as jnp
from jax import lax
from jax.experimental import pallas as pl
from jax.experimental.pallas import tpu as pltpu
from jax.experimental.pallas import tpu_sc as plsc

VOCAB_N = 1000000
DIM = 64
CTX = 200
BLK = 8000
NB = VOCAB_N // BLK

SC_CORES = 2
SC_SUBCORES = 16
SC_WORKERS = SC_CORES * SC_SUBCORES
B_PER_W = 16
PAD_B = SC_WORKERS * B_PER_W
N_SLABS = VOCAB_N // 8
NROWS = PAD_B * 8
INV_CTX = 1.0 / CTX


def _sc_gather_body(table_hbm, idx_hbm, psum_hbm, idx_v, rows_v, acc_v, sem):
    wid = lax.axis_index("s") * SC_CORES + lax.axis_index("c")
    base = wid * B_PER_W
    pltpu.sync_copy(idx_hbm.at[pl.ds(base, B_PER_W)], idx_v)
    iv = idx_v[...]
    lane = lax.iota(jnp.int32, 16)

    copies = []
    for t in range(B_PER_W):
        row_t = jnp.max(jnp.where(lane == t, iv, jnp.int32(-1)))
        cp = pltpu.make_async_copy(
            table_hbm.at[pl.ds(row_t, 1)], rows_v.at[pl.ds(t, 1)], sem)
        cp.start()
        copies.append(cp)
    for cp in copies:
        cp.wait()

    accs = [jnp.zeros((16,), jnp.float32) for _ in range(DIM // 16)]
    for t in range(B_PER_W):
        wt = jnp.where((base + t) < CTX, jnp.float32(1.0), jnp.float32(0.0))
        for c in range(DIM // 16):
            accs[c] = accs[c] + rows_v[t, pl.ds(16 * c, 16)] * wt
    for c in range(DIM // 16):
        acc_v[0, pl.ds(16 * c, 16)] = accs[c]
    pltpu.sync_copy(acc_v, psum_hbm.at[pl.ds(wid, 1)])


@functools.lru_cache(maxsize=1)
def _sc_gather_call():
    return pl.kernel(
        _sc_gather_body,
        mesh=plsc.VectorSubcoreMesh(core_axis_name="c", subcore_axis_name="s"),
        compiler_params=pltpu.CompilerParams(needs_layout_passes=False),
        out_type=jax.ShapeDtypeStruct((SC_WORKERS, DIM), jnp.float32),
        scratch_types=[
            pltpu.VMEM((B_PER_W,), jnp.int32),
            pltpu.VMEM((B_PER_W, DIM), jnp.float32),
            pltpu.VMEM((1, DIM), jnp.float32),
            pltpu.SemaphoreType.DMA,
        ],
    )


def _cbow_body(psum_ref, wm_ref, b_ref, out_ref, m_ref, s_ref):
    i = pl.program_id(0)

    @pl.when(i == 0)
    def _init():
        m_ref[0] = -jnp.inf
        s_ref[0] = 0.0

    v = jnp.sum(psum_ref[...], axis=0, keepdims=True) * INV_CTX

    wb = wm_ref[0]
    xb = lax.dot_general(
        v, wb, (((1,), (1,)), ((), ())), preferred_element_type=jnp.float32
    )
    xb = xb + b_ref[0]
    out_ref[pl.ds(i, 1), :] = xb

    bm = jnp.max(xb)
    m_old = m_ref[0]
    m_new = jnp.maximum(m_old, bm)
    s_ref[0] = s_ref[0] * jnp.exp(m_old - m_new) + jnp.sum(jnp.exp(xb - m_new))
    m_ref[0] = m_new

    @pl.when(i == NB - 1)
    def _fin():
        lse = m_ref[0] + jnp.log(s_ref[0])
        out_ref[...] = out_ref[...] - lse


_cbow_call = pl.pallas_call(
    _cbow_body,
    grid=(NB,),
    in_specs=[
        pl.BlockSpec((SC_WORKERS, DIM), lambda i: (0, 0)),
        pl.BlockSpec((1, BLK, DIM), lambda i: (i, 0, 0)),
        pl.BlockSpec((1, 1, BLK), lambda i: (i, 0, 0)),
    ],
    out_specs=pl.BlockSpec((NB, BLK), lambda i: (0, 0)),
    out_shape=jax.ShapeDtypeStruct((NB, BLK), jnp.float32),
    scratch_shapes=[
        pltpu.SMEM((1,), jnp.float32),
        pltpu.SMEM((1,), jnp.float32),
    ],
    compiler_params=pltpu.CompilerParams(
        dimension_semantics=("arbitrary",),
    ),
)


def kernel(inputs, table, W, b):
    idx = inputs.astype(jnp.int32)
    idxp = jnp.concatenate([idx, jnp.zeros((PAD_B - CTX,), jnp.int32)])
    psum = _sc_gather_call()(table, idxp)
    out2d = _cbow_call(
        psum,
        W.reshape(NB, BLK, DIM),
        b.reshape(NB, 1, BLK),
    )
    return out2d.reshape(1, VOCAB_N)


# speedup vs baseline: 1.0005x; 1.0005x over previous
import functools

import jax
import jax.numpy as jnp
from jax import lax
from jax.experimental import pallas as pl
from jax.experimental.pallas import tpu as pltpu
from jax.experimental.pallas import tpu_sc as plsc

VOCAB_N = 1000000
DIM = 64
CTX = 200
BLK = 8000
NB = VOCAB_N // BLK

SC_CORES = 2
SC_SUBCORES = 16
SC_WORKERS = SC_CORES * SC_SUBCORES
B_PER_W = 16
PAD_B = SC_WORKERS * B_PER_W
N_SLABS = VOCAB_N // 8
NROWS = PAD_B * 8
INV_CTX = 1.0 / CTX


def _sc_gather_body(table_hbm, idx_hbm, psum_hbm, idx_v, rows_v, acc_v, sem):
    wid = lax.axis_index("s") * SC_CORES + lax.axis_index("c")
    base = wid * B_PER_W
    pltpu.sync_copy(idx_hbm.at[pl.ds(base, B_PER_W)], idx_v)
    iv = idx_v[...]
    lane = lax.iota(jnp.int32, 16)

    copies = []
    for t in range(B_PER_W):
        row_t = jnp.max(jnp.where(lane == t, iv, jnp.int32(-1)))
        cp = pltpu.make_async_copy(
            table_hbm.at[pl.ds(row_t, 1)], rows_v.at[pl.ds(t, 1)], sem)
        cp.start()
        copies.append(cp)
    for cp in copies:
        cp.wait()

    accs = [jnp.zeros((16,), jnp.float32) for _ in range(DIM // 16)]
    for t in range(B_PER_W):
        wt = jnp.where((base + t) < CTX, jnp.float32(1.0), jnp.float32(0.0))
        for c in range(DIM // 16):
            accs[c] = accs[c] + rows_v[t, pl.ds(16 * c, 16)] * wt
    for c in range(DIM // 16):
        acc_v[0, pl.ds(16 * c, 16)] = accs[c]
    pltpu.sync_copy(acc_v, psum_hbm.at[pl.ds(wid, 1)])


@functools.lru_cache(maxsize=1)
def _sc_gather_call():
    return pl.kernel(
        _sc_gather_body,
        mesh=plsc.VectorSubcoreMesh(core_axis_name="c", subcore_axis_name="s"),
        compiler_params=pltpu.CompilerParams(
            needs_layout_passes=False, use_tc_tiling_on_sc=True),
        out_type=jax.ShapeDtypeStruct((SC_WORKERS, DIM), jnp.float32),
        scratch_types=[
            pltpu.VMEM((B_PER_W,), jnp.int32),
            pltpu.VMEM((B_PER_W, DIM), jnp.float32),
            pltpu.VMEM((1, DIM), jnp.float32),
            pltpu.SemaphoreType.DMA,
        ],
    )


def _cbow_body(psum_ref, wm_ref, b_ref, out_ref, m_ref, s_ref):
    i = pl.program_id(0)

    @pl.when(i == 0)
    def _init():
        m_ref[0] = -jnp.inf
        s_ref[0] = 0.0

    v = jnp.sum(psum_ref[...], axis=0, keepdims=True) * INV_CTX

    wb = wm_ref[0]
    xb = lax.dot_general(
        v, wb, (((1,), (1,)), ((), ())), preferred_element_type=jnp.float32
    )
    xb = xb + b_ref[0]
    out_ref[pl.ds(i, 1), :] = xb

    bm = jnp.max(xb)
    m_old = m_ref[0]
    m_new = jnp.maximum(m_old, bm)
    s_ref[0] = s_ref[0] * jnp.exp(m_old - m_new) + jnp.sum(jnp.exp(xb - m_new))
    m_ref[0] = m_new

    @pl.when(i == NB - 1)
    def _fin():
        lse = m_ref[0] + jnp.log(s_ref[0])
        out_ref[...] = out_ref[...] - lse


_cbow_call = pl.pallas_call(
    _cbow_body,
    grid=(NB,),
    in_specs=[
        pl.BlockSpec((SC_WORKERS, DIM), lambda i: (0, 0)),
        pl.BlockSpec((1, BLK, DIM), lambda i: (i, 0, 0)),
        pl.BlockSpec((1, 1, BLK), lambda i: (i, 0, 0)),
    ],
    out_specs=pl.BlockSpec((NB, BLK), lambda i: (0, 0)),
    out_shape=jax.ShapeDtypeStruct((NB, BLK), jnp.float32),
    scratch_shapes=[
        pltpu.SMEM((1,), jnp.float32),
        pltpu.SMEM((1,), jnp.float32),
    ],
    compiler_params=pltpu.CompilerParams(
        dimension_semantics=("arbitrary",),
    ),
)


def kernel(inputs, table, W, b):
    idx = inputs.astype(jnp.int32)
    idxp = jnp.concatenate([idx, jnp.zeros((PAD_B - CTX,), jnp.int32)])
    psum = _sc_gather_call()(table, idxp)
    out2d = _cbow_call(
        psum,
        W.reshape(NB, BLK, DIM),
        b.reshape(NB, 1, BLK),
    )
    return out2d.reshape(1, VOCAB_N)


# speedup vs baseline: 5.5071x; 5.5043x over previous
import jax
import jax.numpy as jnp
from jax import lax
from jax.experimental import pallas as pl
from jax.experimental.pallas import tpu as pltpu

VOCAB_N = 1000000
DIM = 64
CTX = 200
CTX_PAD = 256
BLKV = 16384
NB = pl.cdiv(VOCAB_N, BLKV)
SBLK = 131072
NSUB = pl.cdiv(VOCAB_N, SBLK)
INV_CTX = 1.0 / CTX


def _cbow_body(idx_ref, tbl_ref, wt_ref, b_ref, out_ref, lse_ref,
               cols, vscr, m_ref, s_ref, sem):
    i = pl.program_id(0)

    @pl.when(i == 0)
    def _gather_and_mean():
        m_ref[0] = -jnp.inf
        s_ref[0] = 0.0
        cps = []
        for t in range(CTX):
            c_al = pl.multiple_of(
                lax.shift_left(lax.shift_right_logical(idx_ref[t], 7), 7),
                128)
            cp = pltpu.make_async_copy(
                tbl_ref.at[:, pl.ds(c_al, 128)], cols.at[t], sem)
            cp.start()
            cps.append(cp)
        for cp in cps:
            cp.wait()
        lane = lax.broadcasted_iota(jnp.int32, (DIM, 128), 1)
        accs = [jnp.zeros((DIM, 128), jnp.float32) for _ in range(4)]
        for t in range(CTX):
            p_t = jnp.bitwise_and(idx_ref[t], 127)
            accs[t % 4] = accs[t % 4] + jnp.where(lane == p_t, cols[t], 0.0)
        acc = (accs[0] + accs[1]) + (accs[2] + accs[3])
        vscr[:, 0:1] = jnp.sum(acc, axis=1, keepdims=True) * INV_CTX

    v = vscr[:, 0:1]
    xb = lax.dot_general(
        v, wt_ref[...], (((0,), (0,)), ((), ())),
        preferred_element_type=jnp.float32,
    )
    xb = xb + b_ref[...]
    out_ref[...] = xb

    col = lax.broadcasted_iota(jnp.int32, (1, BLKV), 1) + i * BLKV
    xm = jnp.where(col < VOCAB_N, xb, -jnp.inf)
    bm = jnp.max(xm)
    m_old = m_ref[0]
    m_new = jnp.maximum(m_old, bm)
    s_ref[0] = s_ref[0] * jnp.exp(m_old - m_new) + jnp.sum(jnp.exp(xm - m_new))
    m_ref[0] = m_new

    @pl.when(i == NB - 1)
    def _finish():
        lse_ref[...] = jnp.full((1, 1), m_ref[0] + jnp.log(s_ref[0]),
                                jnp.float32)


_cbow_call = pl.pallas_call(
    _cbow_body,
    grid_spec=pltpu.PrefetchScalarGridSpec(
        num_scalar_prefetch=1,
        grid=(NB,),
        in_specs=[
            pl.BlockSpec(memory_space=pl.ANY),
            pl.BlockSpec((DIM, BLKV), lambda i, idx_ref: (0, i)),
            pl.BlockSpec((1, BLKV), lambda i, idx_ref: (0, i)),
        ],
        out_specs=[
            pl.BlockSpec((1, BLKV), lambda i, idx_ref: (0, i)),
            pl.BlockSpec((1, 1), lambda i, idx_ref: (0, 0)),
        ],
        scratch_shapes=[
            pltpu.VMEM((CTX, DIM, 128), jnp.float32),
            pltpu.VMEM((DIM, 128), jnp.float32),
            pltpu.SMEM((1,), jnp.float32),
            pltpu.SMEM((1,), jnp.float32),
            pltpu.SemaphoreType.DMA,
        ],
    ),
    out_shape=[
        jax.ShapeDtypeStruct((1, VOCAB_N), jnp.float32),
        jax.ShapeDtypeStruct((1, 1), jnp.float32),
    ],
    compiler_params=pltpu.CompilerParams(
        dimension_semantics=("arbitrary",),
    ),
)


def _sub_body(x_ref, lse_ref, o_ref):
    o_ref[...] = x_ref[...] - lse_ref[0, 0]


_sub_call = pl.pallas_call(
    _sub_body,
    grid=(NSUB,),
    in_specs=[
        pl.BlockSpec((1, SBLK), lambda i: (0, i)),
        pl.BlockSpec(memory_space=pltpu.SMEM),
    ],
    out_specs=pl.BlockSpec((1, SBLK), lambda i: (0, i)),
    out_shape=jax.ShapeDtypeStruct((1, VOCAB_N), jnp.float32),
    compiler_params=pltpu.CompilerParams(
        dimension_semantics=("arbitrary",),
    ),
)


def kernel(inputs, table, W, b):
    idx = inputs.astype(jnp.int32)
    logits, lse = _cbow_call(idx, table.T, W.T, b.reshape(1, VOCAB_N))
    return _sub_call(logits, lse)


# speedup vs baseline: 6.5639x; 1.1919x over previous
import jax
import jax.numpy as jnp
from jax import lax
from jax.experimental import pallas as pl
from jax.experimental.pallas import tpu as pltpu

VOCAB_N = 1000000
DIM = 64
CTX = 200
CTX_PAD = 256
BLKV = 32768
NB = pl.cdiv(VOCAB_N, BLKV)
SBLK = 131072
NSUB = pl.cdiv(VOCAB_N, SBLK)
INV_CTX = 1.0 / CTX


def _cbow_body(idx_ref, tbl_ref, wt_ref, b_ref, out_ref, lse_ref,
               cols, vscr, m_ref, s_ref, sem):
    i = pl.program_id(0)

    @pl.when(i == 0)
    def _gather_and_mean():
        m_ref[0] = -jnp.inf
        s_ref[0] = 0.0
        cps = []
        for t in range(CTX):
            c_al = pl.multiple_of(
                lax.shift_left(lax.shift_right_logical(idx_ref[t], 7), 7),
                128)
            cp = pltpu.make_async_copy(
                tbl_ref.at[:, pl.ds(c_al, 128)], cols.at[t], sem)
            cp.start()
            cps.append(cp)
        for cp in cps:
            cp.wait()
        lane = lax.broadcasted_iota(jnp.int32, (DIM, 128), 1)
        accs = [jnp.zeros((DIM, 128), jnp.float32) for _ in range(4)]
        for t in range(CTX):
            p_t = jnp.bitwise_and(idx_ref[t], 127)
            accs[t % 4] = accs[t % 4] + jnp.where(lane == p_t, cols[t], 0.0)
        acc = (accs[0] + accs[1]) + (accs[2] + accs[3])
        vscr[:, 0:1] = jnp.sum(acc, axis=1, keepdims=True) * INV_CTX

    v = vscr[:, 0:1]
    xb = lax.dot_general(
        v, wt_ref[...], (((0,), (0,)), ((), ())),
        preferred_element_type=jnp.float32,
    )
    xb = xb + b_ref[...]
    out_ref[...] = xb

    col = lax.broadcasted_iota(jnp.int32, (1, BLKV), 1) + i * BLKV
    xm = jnp.where(col < VOCAB_N, xb, -jnp.inf)
    bm = jnp.max(xm)
    m_old = m_ref[0]
    m_new = jnp.maximum(m_old, bm)
    s_ref[0] = s_ref[0] * jnp.exp(m_old - m_new) + jnp.sum(jnp.exp(xm - m_new))
    m_ref[0] = m_new

    @pl.when(i == NB - 1)
    def _finish():
        lse_ref[...] = jnp.full((1, 1), m_ref[0] + jnp.log(s_ref[0]),
                                jnp.float32)


_cbow_call = pl.pallas_call(
    _cbow_body,
    grid_spec=pltpu.PrefetchScalarGridSpec(
        num_scalar_prefetch=1,
        grid=(NB,),
        in_specs=[
            pl.BlockSpec(memory_space=pl.ANY),
            pl.BlockSpec((DIM, BLKV), lambda i, idx_ref: (0, i)),
            pl.BlockSpec((1, BLKV), lambda i, idx_ref: (0, i)),
        ],
        out_specs=[
            pl.BlockSpec((1, BLKV), lambda i, idx_ref: (0, i)),
            pl.BlockSpec((1, 1), lambda i, idx_ref: (0, 0)),
        ],
        scratch_shapes=[
            pltpu.VMEM((CTX, DIM, 128), jnp.float32),
            pltpu.VMEM((DIM, 128), jnp.float32),
            pltpu.SMEM((1,), jnp.float32),
            pltpu.SMEM((1,), jnp.float32),
            pltpu.SemaphoreType.DMA,
        ],
    ),
    out_shape=[
        jax.ShapeDtypeStruct((1, VOCAB_N), jnp.float32),
        jax.ShapeDtypeStruct((1, 1), jnp.float32),
    ],
    compiler_params=pltpu.CompilerParams(
        dimension_semantics=("arbitrary",),
    ),
)


def _sub_body(x_ref, lse_ref, o_ref):
    o_ref[...] = x_ref[...] - lse_ref[0, 0]


_sub_call = pl.pallas_call(
    _sub_body,
    grid=(NSUB,),
    in_specs=[
        pl.BlockSpec((1, SBLK), lambda i: (0, i)),
        pl.BlockSpec(memory_space=pltpu.SMEM),
    ],
    out_specs=pl.BlockSpec((1, SBLK), lambda i: (0, i)),
    out_shape=jax.ShapeDtypeStruct((1, VOCAB_N), jnp.float32),
    compiler_params=pltpu.CompilerParams(
        dimension_semantics=("arbitrary",),
    ),
)


def kernel(inputs, table, W, b):
    idx = inputs.astype(jnp.int32)
    logits, lse = _cbow_call(idx, table.T, W.T, b.reshape(1, VOCAB_N))
    return _sub_call(logits, lse)
